# Initial kernel scaffold; baseline (speedup 1.0000x reference)
import jax
import jax.numpy as jnp
from jax.experimental import pallas as pl


def kernel(x, w1, b1, w2, b2, fw1, fb1, fw2, fb2):
    raise NotImplementedError("write your pallas kernel here")


# speedup vs baseline: 14.1209x; 14.1209x over previous
import numpy as np
import jax
import jax.numpy as jnp
from jax.experimental import pallas as pl
from jax.experimental.pallas import tpu as pltpu

C_IN = 3
D1 = 30
D2 = 60
CLASSES = 43
HID = 512
IMG = 32
P1 = 16
C2OUT = 14
P2 = 7
L1 = C_IN * IMG
N1 = D1 * IMG
L2 = D1 * P1
N2 = D2 * C2OUT
L3 = D2 * P2

_E1 = np.stack([np.eye(IMG, IMG, k=1 - kw, dtype=np.float32)
                for kw in range(3)])
_E2 = np.stack([np.eye(P1, C2OUT, k=-kw, dtype=np.float32)
                for kw in range(3)])


def _net_kernel(x_ref, w1t_ref, b1r_ref, w2t_ref, b2r_ref,
                fw1_ref, fb1_ref, fw2_ref, fb2_ref, o_ref,
                xs_ref, h1_ref, c2_ref):
    tn = x_ref.shape[0]

    zrow = jnp.zeros((tn, 1, L1), jnp.float32)
    xs_ref[:, 0:1, :] = zrow
    xs_ref[:, IMG + 1:IMG + 2, :] = zrow
    for c in range(C_IN):
        xs_ref[:, 1:1 + IMG, c * IMG:(c + 1) * IMG] = x_ref[:, c, :, :]

    acc1 = jnp.zeros((tn * IMG, N1), jnp.float32)
    for kh in range(3):
        rows = xs_ref[:, kh:kh + IMG, :].reshape(tn * IMG, L1)
        acc1 = acc1 + jnp.dot(rows, w1t_ref[kh],
                              preferred_element_type=jnp.float32)
    a1 = acc1.reshape(tn, IMG, N1)
    pw = jnp.maximum(a1[:, :, :L2], a1[:, :, L2:])
    for hh in range(P1):
        h1_ref[:, hh, :] = jnp.maximum(
            jnp.maximum(pw[:, 2 * hh, :], pw[:, 2 * hh + 1, :])
            + b1r_ref[0], 0.0)

    acc2 = jnp.zeros((tn * C2OUT, N2), jnp.float32)
    for kh in range(3):
        rows = h1_ref[:, kh:kh + C2OUT, :].reshape(tn * C2OUT, L2)
        acc2 = acc2 + jnp.dot(rows, w2t_ref[kh],
                              preferred_element_type=jnp.float32)
    a2 = acc2.reshape(tn, C2OUT, N2)
    qw = jnp.maximum(a2[:, :, :L3], a2[:, :, L3:])
    for hh in range(P2):
        c2_ref[:, hh, :] = jnp.maximum(
            jnp.maximum(qw[:, 2 * hh, :], qw[:, 2 * hh + 1, :])
            + b2r_ref[0], 0.0)

    hacc = jnp.zeros((tn, HID), jnp.float32)
    for hh in range(P2):
        hacc = hacc + jnp.dot(c2_ref[:, hh, :], fw1_ref[hh],
                              preferred_element_type=jnp.float32)
    h = jnp.maximum(hacc + fb1_ref[...], 0.0)

    y = jnp.dot(h, fw2_ref[...], preferred_element_type=jnp.float32)
    y = jnp.maximum(y + fb2_ref[...], 0.0)
    m = jnp.max(y, axis=1, keepdims=True)
    z = y - m
    lse = jnp.log(jnp.sum(jnp.exp(z), axis=1, keepdims=True))
    o_ref[...] = z - lse


def _tile(n):
    for t in (128, 64, 32, 16, 8, 4, 2, 1):
        if n % t == 0 and t <= n:
            return t
    return 1


def kernel(x, w1, b1, w2, b2, fw1, fb1, fw2, fb2):
    n = x.shape[0]
    tn = _tile(n)
    x = x.astype(jnp.float32)

    e1 = jnp.asarray(_E1)
    e2 = jnp.asarray(_E2)
    w1t = jnp.einsum('hkcd,kij->hcidj', w1, e1).reshape(3, L1, N1)
    w2t = jnp.einsum('hkcd,kij->hcidj', w2, e2).reshape(3, L2, N2)
    w1t = w1t.reshape(3, L1, D1, P1, 2).transpose(0, 1, 4, 2, 3)
    w1t = w1t.reshape(3, L1, N1)
    w2t = w2t.reshape(3, L2, D2, P2, 2).transpose(0, 1, 4, 2, 3)
    w2t = w2t.reshape(3, L2, N2)
    b1r = jnp.repeat(b1.reshape(D1, 1), P1, axis=1).reshape(1, L2)
    b2r = jnp.repeat(b2.reshape(D2, 1), P2, axis=1).reshape(1, L3)
    fw1o = jnp.transpose(fw1, (0, 2, 1, 3)).reshape(P2, L3, HID)

    out = pl.pallas_call(
        _net_kernel,
        out_shape=jax.ShapeDtypeStruct((n, CLASSES), jnp.float32),
        grid=(n // tn,),
        in_specs=[
            pl.BlockSpec((tn, C_IN, IMG, IMG), lambda i: (i, 0, 0, 0)),
            pl.BlockSpec((3, L1, N1), lambda i: (0, 0, 0)),
            pl.BlockSpec((1, L2), lambda i: (0, 0)),
            pl.BlockSpec((3, L2, N2), lambda i: (0, 0, 0)),
            pl.BlockSpec((1, L3), lambda i: (0, 0)),
            pl.BlockSpec((P2, L3, HID), lambda i: (0, 0, 0)),
            pl.BlockSpec((1, HID), lambda i: (0, 0)),
            pl.BlockSpec((HID, CLASSES), lambda i: (0, 0)),
            pl.BlockSpec((1, CLASSES), lambda i: (0, 0)),
        ],
        out_specs=pl.BlockSpec((tn, CLASSES), lambda i: (i, 0)),
        scratch_shapes=[pltpu.VMEM((tn, IMG + 2, L1), jnp.float32),
                        pltpu.VMEM((tn, P1, L2), jnp.float32),
                        pltpu.VMEM((tn, P2, L3), jnp.float32)],
        compiler_params=pltpu.CompilerParams(
            dimension_semantics=("parallel",),
            vmem_limit_bytes=56 * 1024 * 1024),
    )(x, w1t, b1r, w2t, b2r, fw1o, fb1, fw2, fb2)
    return out
